# baseline (speedup 1.0000x reference)
import jax
import jax.numpy as jnp
from jax import lax
from jax.experimental import pallas as pl
from jax.experimental.pallas import tpu as pltpu
from jax.experimental.pallas import tpu_sc as plsc

_D = 128
_NROWS = 12
_S1 = 128 * 64
_S2 = 64


def _body(i1_hbm, i2_hbm, i3_hbm, xflat_hbm, out_hbm, i1_s, i2_s, i3_s, sem):
    c1 = pltpu.make_async_copy(i1_hbm, i1_s, sem)
    c2 = pltpu.make_async_copy(i2_hbm, i2_s, sem)
    c3 = pltpu.make_async_copy(i3_hbm, i3_s, sem)
    c1.start()
    c2.start()
    c3.start()
    c1.wait()
    c2.wait()
    c3.wait()
    copies = []
    for i in range(_NROWS):
        flat = i1_s[i // 3] * _S1 + i2_s[i % 3] * _S2 + i3_s[i]
        cp = pltpu.make_async_copy(
            xflat_hbm.at[pl.ds(flat, 1)], out_hbm.at[pl.ds(i, 1)], sem)
        cp.start()
        copies.append(cp)
    for cp in copies:
        cp.wait()


def kernel(x, index1, index2, index3):
    xflat = x.reshape(-1, _D)
    mesh = plsc.ScalarSubcoreMesh(axis_name="c", num_cores=1)
    out = pl.kernel(
        _body,
        out_type=jax.ShapeDtypeStruct((_NROWS, _D), jnp.float32),
        mesh=mesh,
        compiler_params=pltpu.CompilerParams(needs_layout_passes=False),
        scratch_types=[
            pltpu.SMEM((4,), jnp.int32),
            pltpu.SMEM((3,), jnp.int32),
            pltpu.SMEM((_NROWS,), jnp.int32),
            pltpu.SemaphoreType.DMA,
        ],
    )(index1.reshape(4).astype(jnp.int32),
      index2.reshape(3).astype(jnp.int32),
      index3.reshape(_NROWS).astype(jnp.int32),
      xflat)
    return out.reshape(4, 3, _D)
